# Initial kernel scaffold; baseline (speedup 1.0000x reference)
import jax
import jax.numpy as jnp
from jax.experimental import pallas as pl


def kernel(x, edge_index, W1, a_src1, a_dst1, b1, W2, a_src2, a_dst2, b2):
    raise NotImplementedError("write your pallas kernel here")


# speedup vs baseline: 22.3870x; 22.3870x over previous
import dataclasses
import functools

import jax
import jax.numpy as jnp
from jax import lax
from jax.experimental import pallas as pl
from jax.experimental.pallas import tpu as pltpu
from jax.experimental.pallas import tpu_sc as plsc

N = 10000
E = 320000
F_IN = 128
HID = 64
HEADS = 8
NCLS = 40

NB = 1000
GRID = N // NB
NSC = 2
NTILE = 16
LANES = 16
EPC = E // NSC
EPT = EPC // NTILE
CH = 80
NP = 10240
STRIPE = NP // NTILE

_MESH = plsc.VectorSubcoreMesh(core_axis_name="c", subcore_axis_name="s")

_SC_PARAMS = pltpu.CompilerParams()
for _f, _v in (("needs_layout_passes", False), ("use_tc_tiling_on_sc", False)):
    if _f in pltpu.CompilerParams.__dataclass_fields__:
        _SC_PARAMS = dataclasses.replace(_SC_PARAMS, **{_f: _v})


def _f32(*shape):
    return jax.ShapeDtypeStruct(shape, jnp.float32)


def _tc1_body(x_ref, w1_ref, abd_ref, h0, h1, h2, h3, a1_ref, amax_ref):
    i = pl.program_id(0)
    h = jnp.dot(x_ref[...], w1_ref[...], preferred_element_type=jnp.float32)
    h0[...] = h[:, 0:128]
    h1[...] = h[:, 128:256]
    h2[...] = h[:, 256:384]
    h3[...] = h[:, 384:512]
    a1 = jnp.dot(h, abd_ref[...], preferred_element_type=jnp.float32)
    a1_ref[...] = a1
    cm = jnp.max(a1, axis=0, keepdims=True)

    @pl.when(i == 0)
    def _():
        amax_ref[...] = cm

    @pl.when(i > 0)
    def _():
        amax_ref[...] = jnp.maximum(amax_ref[...], cm)


def _tc1(x, W1, Abd1):
    outs = [_f32(N, 128)] * 4 + [_f32(N, 16), _f32(1, 16)]
    return pl.pallas_call(
        _tc1_body,
        grid=(GRID,),
        in_specs=[
            pl.BlockSpec((NB, F_IN), lambda i: (i, 0)),
            pl.BlockSpec((F_IN, 512), lambda i: (0, 0)),
            pl.BlockSpec((512, 16), lambda i: (0, 0)),
        ],
        out_specs=[pl.BlockSpec((NB, 128), lambda i: (i, 0))] * 4
        + [
            pl.BlockSpec((NB, 16), lambda i: (i, 0)),
            pl.BlockSpec((1, 16), lambda i: (0, 0)),
        ],
        out_shape=outs,
    )(x, W1, Abd1)


def _sc1_body(h0, h1, h2, h3, a1, m1t, src, dst, z128, z16,
              n0, n1, n2, n3, denp,
              idx_s, idx_d, hs, as_, ad, ee, m1v,
              num_acc, den_acc, sem):
    cid = lax.axis_index("c")
    sid = lax.axis_index("s")
    row0 = sid * STRIPE
    stripe = pl.ds(row0, STRIPE)
    base = cid * EPC + sid * EPT

    pltpu.sync_copy(m1t, m1v)
    pltpu.sync_copy(z16.at[stripe], den_acc.at[stripe])

    htabs = [h0, h1, h2, h3]
    ntabs = [n0, n1, n2, n3]
    iota = lax.broadcasted_iota(jnp.int32, (LANES,), 0)

    for s in range(4):
        htab = htabs[s]
        hcol0 = jnp.full((LANES,), 2 * s, jnp.int32)
        hcol1 = jnp.full((LANES,), 2 * s + 1, jnp.int32)
        dcol0 = jnp.full((LANES,), 8 + 2 * s, jnp.int32)
        dcol1 = jnp.full((LANES,), 8 + 2 * s + 1, jnp.int32)
        mv0 = m1v[2 * s, :]
        mv1 = m1v[2 * s + 1, :]

        pltpu.sync_copy(z128.at[stripe], num_acc.at[stripe])
        pltpu.sync_copy(z16.at[pl.ds(0, CH)], ee)
        plsc.subcore_barrier()

        @pl.loop(0, EPT, step=CH)
        def _chunk(c):
            lo = base + c
            pltpu.sync_copy(src.at[pl.ds(lo, CH)], idx_s)
            pltpu.sync_copy(dst.at[pl.ds(lo, CH)], idx_d)
            cp1 = pltpu.async_copy(htab.at[idx_s], hs, sem)
            cp2 = pltpu.async_copy(a1.at[idx_s], as_, sem)
            cp3 = pltpu.async_copy(a1.at[idx_d], ad, sem)
            cp1.wait()
            cp2.wait()
            cp3.wait()

            @pl.loop(0, CH, step=LANES)
            def _group(g):
                rows = iota + g
                s0 = plsc.load_gather(as_, [rows, hcol0])
                d0 = plsc.load_gather(ad, [rows, dcol0])
                s1 = plsc.load_gather(as_, [rows, hcol1])
                d1 = plsc.load_gather(ad, [rows, dcol1])
                e0 = s0 + d0
                e1 = s1 + d1
                e0 = jnp.maximum(e0, 0.2 * e0)
                e1 = jnp.maximum(e1, 0.2 * e1)
                ee0 = jnp.exp(e0 - mv0)
                ee1 = jnp.exp(e1 - mv1)
                plsc.store_scatter(ee, [rows, hcol0], ee0)
                plsc.store_scatter(ee, [rows, hcol1], ee1)
                for k in range(LANES):
                    row = g + k
                    rv = jnp.full((LANES,), row, jnp.int32)
                    a0 = plsc.load_gather(ee, [rv, hcol0])
                    a1v = plsc.load_gather(ee, [rv, hcol1])
                    for v in range(4):
                        sl = pl.ds(v * LANES, LANES)
                        hs[row, sl] = hs[row, sl] * a0
                    for v in range(4, 8):
                        sl = pl.ds(v * LANES, LANES)
                        hs[row, sl] = hs[row, sl] * a1v

            pltpu.sync_copy(hs, num_acc.at[idx_d], add=True)
            pltpu.sync_copy(ee, den_acc.at[idx_d], add=True)

        plsc.subcore_barrier()
        pltpu.sync_copy(num_acc.at[stripe],
                        ntabs[s].at[pl.ds(cid * NP + row0, STRIPE)])

    pltpu.sync_copy(den_acc.at[stripe],
                    denp.at[pl.ds(cid * NP + row0, STRIPE)])


def _sc1(h0, h1, h2, h3, a1, m1t, src, dst, z128, z16):
    k = pl.kernel(
        _sc1_body,
        mesh=_MESH,
        out_type=[_f32(NSC * NP, 128)] * 4 + [_f32(NSC * NP, 16)],
        scratch_types=[
            pltpu.VMEM((CH,), jnp.int32),
            pltpu.VMEM((CH,), jnp.int32),
            pltpu.VMEM((CH, 128), jnp.float32),
            pltpu.VMEM((CH, 16), jnp.float32),
            pltpu.VMEM((CH, 16), jnp.float32),
            pltpu.VMEM((CH, 16), jnp.float32),
            pltpu.VMEM((HEADS, 16), jnp.float32),
            pltpu.VMEM_SHARED((NP, 128), jnp.float32),
            pltpu.VMEM_SHARED((NP, 16), jnp.float32),
            pltpu.SemaphoreType.DMA,
        ],
        compiler_params=_SC_PARAMS,
    )
    return k(h0, h1, h2, h3, a1, m1t, src, dst, z128, z16)


def _tc2_body(n0, n1, n2, n3, denp_ref, b1_ref, w2p_ref, r16_ref,
              h2p_ref, d2_ref, a2max_ref):
    i = pl.program_id(0)
    num = jnp.concatenate(
        [n0[0] + n0[1], n1[0] + n1[1], n2[0] + n2[1], n3[0] + n3[1]], axis=1)
    den = denp_ref[0] + denp_ref[1]
    dinv = jnp.where(den > 0.0, 1.0 / den, 0.0)
    scale = jnp.dot(dinv, r16_ref[...], preferred_element_type=jnp.float32)
    out1 = num * scale + b1_ref[...]
    act = jnp.where(out1 > 0.0, out1, jnp.exp(out1) - 1.0)
    h2p = jnp.dot(act, w2p_ref[...], preferred_element_type=jnp.float32)
    h2p_ref[...] = h2p
    d2_ref[...] = h2p[:, 32:48]
    cm = jnp.max(h2p, axis=0, keepdims=True)

    @pl.when(i == 0)
    def _():
        a2max_ref[...] = cm

    @pl.when(i > 0)
    def _():
        a2max_ref[...] = jnp.maximum(a2max_ref[...], cm)


def _tc2(n0, n1, n2, n3, denp, b1r, W2p, R16):
    outs = [_f32(N, 48), _f32(N, 16), _f32(1, 48)]
    return pl.pallas_call(
        _tc2_body,
        grid=(GRID,),
        in_specs=[pl.BlockSpec((2, NB, 128), lambda i: (0, i, 0))] * 4
        + [
            pl.BlockSpec((2, NB, 16), lambda i: (0, i, 0)),
            pl.BlockSpec((1, 512), lambda i: (0, 0)),
            pl.BlockSpec((512, 48), lambda i: (0, 0)),
            pl.BlockSpec((16, 512), lambda i: (0, 0)),
        ],
        out_specs=[
            pl.BlockSpec((NB, 48), lambda i: (i, 0)),
            pl.BlockSpec((NB, 16), lambda i: (i, 0)),
            pl.BlockSpec((1, 48), lambda i: (0, 0)),
        ],
        out_shape=outs,
    )(n0, n1, n2, n3, denp, b1r, W2p, R16)


def _sc2_body(h2p, d2, m2t, src, dst, z48, z16,
              nump2, denp2,
              idx_s, idx_d, hs, ad, ee, m2v,
              num_acc, den_acc, sem):
    cid = lax.axis_index("c")
    sid = lax.axis_index("s")
    row0 = sid * STRIPE
    stripe = pl.ds(row0, STRIPE)
    base = cid * EPC + sid * EPT

    pltpu.sync_copy(m2t, m2v)
    pltpu.sync_copy(z48.at[stripe], num_acc.at[stripe])
    pltpu.sync_copy(z16.at[stripe], den_acc.at[stripe])
    pltpu.sync_copy(z16.at[pl.ds(0, CH)], ee)
    plsc.subcore_barrier()

    iota = lax.broadcasted_iota(jnp.int32, (LANES,), 0)
    col40 = jnp.full((LANES,), 40, jnp.int32)
    col9 = jnp.full((LANES,), 9, jnp.int32)
    col0 = jnp.full((LANES,), 0, jnp.int32)
    mv = m2v[0, :]

    @pl.loop(0, EPT, step=CH)
    def _chunk(c):
        lo = base + c
        pltpu.sync_copy(src.at[pl.ds(lo, CH)], idx_s)
        pltpu.sync_copy(dst.at[pl.ds(lo, CH)], idx_d)
        cp1 = pltpu.async_copy(h2p.at[idx_s], hs, sem)
        cp2 = pltpu.async_copy(d2.at[idx_d], ad, sem)
        cp1.wait()
        cp2.wait()

        @pl.loop(0, CH, step=LANES)
        def _group(g):
            rows = iota + g
            sv = plsc.load_gather(hs, [rows, col40])
            dv = plsc.load_gather(ad, [rows, col9])
            e = sv + dv
            e = jnp.maximum(e, 0.2 * e)
            eev = jnp.exp(e - mv)
            plsc.store_scatter(ee, [rows, col0], eev)
            for k in range(LANES):
                row = g + k
                rv = jnp.full((LANES,), row, jnp.int32)
                a0 = plsc.load_gather(ee, [rv, col0])
                for v in range(3):
                    sl = pl.ds(v * LANES, LANES)
                    hs[row, sl] = hs[row, sl] * a0

        pltpu.sync_copy(hs, num_acc.at[idx_d], add=True)
        pltpu.sync_copy(ee, den_acc.at[idx_d], add=True)

    plsc.subcore_barrier()
    pltpu.sync_copy(num_acc.at[stripe],
                    nump2.at[pl.ds(cid * NP + row0, STRIPE)])
    pltpu.sync_copy(den_acc.at[stripe],
                    denp2.at[pl.ds(cid * NP + row0, STRIPE)])


def _sc2(h2p, d2, m2t, src, dst, z48, z16):
    k = pl.kernel(
        _sc2_body,
        mesh=_MESH,
        out_type=[_f32(NSC * NP, 48), _f32(NSC * NP, 16)],
        scratch_types=[
            pltpu.VMEM((CH,), jnp.int32),
            pltpu.VMEM((CH,), jnp.int32),
            pltpu.VMEM((CH, 48), jnp.float32),
            pltpu.VMEM((CH, 16), jnp.float32),
            pltpu.VMEM((CH, 16), jnp.float32),
            pltpu.VMEM((1, 16), jnp.float32),
            pltpu.VMEM_SHARED((NP, 48), jnp.float32),
            pltpu.VMEM_SHARED((NP, 16), jnp.float32),
            pltpu.SemaphoreType.DMA,
        ],
        compiler_params=_SC_PARAMS,
    )
    return k(h2p, d2, m2t, src, dst, z48, z16)


def _tc3_body(nump2_ref, denp2_ref, b2_ref, out_ref):
    num = nump2_ref[0] + nump2_ref[1]
    den = denp2_ref[0] + denp2_ref[1]
    dinv = jnp.where(den > 0.0, 1.0 / den, 0.0)[:, 0:1]
    logits = num[:, 0:NCLS] * dinv + b2_ref[...]
    m = jnp.max(logits, axis=1, keepdims=True)
    z = jnp.exp(logits - m)
    lse = m + jnp.log(jnp.sum(z, axis=1, keepdims=True))
    out_ref[...] = logits - lse


def _tc3(nump2, denp2, b2r):
    return pl.pallas_call(
        _tc3_body,
        grid=(GRID,),
        in_specs=[
            pl.BlockSpec((2, NB, 48), lambda i: (0, i, 0)),
            pl.BlockSpec((2, NB, 16), lambda i: (0, i, 0)),
            pl.BlockSpec((1, NCLS), lambda i: (0, 0)),
        ],
        out_specs=pl.BlockSpec((NB, NCLS), lambda i: (i, 0)),
        out_shape=_f32(N, NCLS),
    )(nump2, denp2, b2r)


@jax.jit
def kernel(x, edge_index, W1, a_src1, a_dst1, b1, W2, a_src2, a_dst2, b2):
    src = edge_index[0]
    dst = edge_index[1]

    eye8 = jnp.eye(HEADS, dtype=jnp.float32)
    asrc_bd = (a_src1[:, :, None] * eye8[:, None, :]).reshape(HEADS * HID, HEADS)
    adst_bd = (a_dst1[:, :, None] * eye8[:, None, :]).reshape(HEADS * HID, HEADS)
    Abd1 = jnp.concatenate([asrc_bd, adst_bd], axis=1)
    W2p = jnp.concatenate(
        [W2, W2 @ a_src2.T, W2 @ a_dst2.T,
         jnp.zeros((HEADS * HID, 6), jnp.float32)], axis=1)
    R8 = jnp.repeat(eye8, HID, axis=1)
    R16 = jnp.concatenate([R8, jnp.zeros((8, 512), jnp.float32)], axis=0)
    b1r = b1.reshape(1, HEADS * HID)
    b2r = b2.reshape(1, NCLS)
    z128 = jnp.zeros((NP, 128), jnp.float32)
    z48 = jnp.zeros((NP, 48), jnp.float32)
    z16 = jnp.zeros((NP, 16), jnp.float32)

    h0, h1, h2, h3, a1, amax1 = _tc1(x, W1, Abd1)
    M1 = jnp.maximum(amax1[0, :8] + amax1[0, 8:], 0.0)
    m1t = jnp.broadcast_to(M1[:, None], (HEADS, 16))

    n0, n1, n2, n3, denp = _sc1(h0, h1, h2, h3, a1, m1t, src, dst, z128, z16)
    n0 = n0.reshape(2, NP, 128)
    n1 = n1.reshape(2, NP, 128)
    n2 = n2.reshape(2, NP, 128)
    n3 = n3.reshape(2, NP, 128)
    denp = denp.reshape(2, NP, 16)

    h2p, d2, a2max = _tc2(n0, n1, n2, n3, denp, b1r, W2p, R16)
    M2 = jnp.maximum(a2max[0, 40] + a2max[0, 41], 0.0)
    m2t = jnp.full((1, 16), M2, jnp.float32)

    nump2, denp2 = _sc2(h2p, d2, m2t, src, dst, z48, z16)
    nump2 = nump2.reshape(2, NP, 48)
    denp2 = denp2.reshape(2, NP, 16)

    return _tc3(nump2, denp2, b2r)


# speedup vs baseline: 44.1678x; 1.9729x over previous
import dataclasses
import functools

import jax
import jax.numpy as jnp
from jax import lax
from jax.experimental import pallas as pl
from jax.experimental.pallas import tpu as pltpu
from jax.experimental.pallas import tpu_sc as plsc

N = 10000
E = 320000
F_IN = 128
HID = 64
HEADS = 8
NCLS = 40

NB = 1000
GRID = N // NB
NSC = 2
NTILE = 16
LANES = 16
EPC = E // NSC
EPT = EPC // NTILE
CH = 80
NP = 10240
STRIPE = NP // NTILE

_MESH = plsc.VectorSubcoreMesh(core_axis_name="c", subcore_axis_name="s")

_SC_PARAMS = pltpu.CompilerParams()
for _f, _v in (("needs_layout_passes", False), ("use_tc_tiling_on_sc", False)):
    if _f in pltpu.CompilerParams.__dataclass_fields__:
        _SC_PARAMS = dataclasses.replace(_SC_PARAMS, **{_f: _v})


def _f32(*shape):
    return jax.ShapeDtypeStruct(shape, jnp.float32)


def _tc1_body(x_ref, w1_ref, abd_ref, *outs):
    hrefs = outs[:HEADS]
    a1_ref, amax_ref = outs[HEADS:]
    i = pl.program_id(0)
    h = jnp.dot(x_ref[...], w1_ref[...], preferred_element_type=jnp.float32)
    for j in range(HEADS):
        hrefs[j][...] = h[:, 64 * j:64 * (j + 1)]
    a1 = jnp.dot(h, abd_ref[...], preferred_element_type=jnp.float32)
    a1_ref[...] = a1
    cm = jnp.max(a1, axis=0, keepdims=True)

    @pl.when(i == 0)
    def _():
        amax_ref[...] = cm

    @pl.when(i > 0)
    def _():
        amax_ref[...] = jnp.maximum(amax_ref[...], cm)


def _tc1(x, W1, Abd1):
    outs = [_f32(N, 64)] * HEADS + [_f32(N, 16), _f32(1, 16)]
    return pl.pallas_call(
        _tc1_body,
        grid=(GRID,),
        in_specs=[
            pl.BlockSpec((NB, F_IN), lambda i: (i, 0)),
            pl.BlockSpec((F_IN, 512), lambda i: (0, 0)),
            pl.BlockSpec((512, 16), lambda i: (0, 0)),
        ],
        out_specs=[pl.BlockSpec((NB, 64), lambda i: (i, 0))] * HEADS
        + [
            pl.BlockSpec((NB, 16), lambda i: (i, 0)),
            pl.BlockSpec((1, 16), lambda i: (0, 0)),
        ],
        out_shape=outs,
    )(x, W1, Abd1)


NCHUNK = EPT // CH
DEPTH = 5


def _drain(dummy_src, dst, sem):
    pltpu.make_async_copy(dummy_src, dst, sem).wait()


def _sc1_body(*args):
    htabs = args[0:HEADS]
    a1, m1t, src, dst, z64, z16 = args[HEADS:HEADS + 6]
    ntabs = args[HEADS + 6:2 * HEADS + 6]
    denp = args[2 * HEADS + 6]
    idx_s, idx_d = args[2 * HEADS + 7:2 * HEADS + 9]
    rest = args[2 * HEADS + 9:]
    hs = rest[0:DEPTH]
    as_ = rest[DEPTH:2 * DEPTH]
    ad = rest[2 * DEPTH:3 * DEPTH]
    ee = rest[3 * DEPTH:4 * DEPTH]
    m1v, num_acc, den_acc = rest[4 * DEPTH:4 * DEPTH + 3]
    sems = rest[4 * DEPTH + 3:]
    sem_g = sems[:DEPTH]
    sem_s = sems[DEPTH:]
    cid = lax.axis_index("c")
    sid = lax.axis_index("s")
    row0 = sid * STRIPE
    stripe = pl.ds(row0, STRIPE)

    pltpu.sync_copy(m1t, m1v)
    pltpu.sync_copy(z16.at[stripe], den_acc.at[stripe])
    pltpu.sync_copy(src.at[cid].at[sid], idx_s)
    pltpu.sync_copy(dst.at[cid].at[sid], idx_d)

    iota = lax.broadcasted_iota(jnp.int32, (LANES,), 0)

    def issue(htab, cc, k):
        pltpu.async_copy(htab.at[idx_s.at[cc]], hs[k], sem_g[k])
        pltpu.async_copy(a1.at[idx_s.at[cc]], as_[k], sem_g[k])
        pltpu.async_copy(a1.at[idx_d.at[cc]], ad[k], sem_g[k])

    for s in range(HEADS):
        htab = htabs[s]
        hcol = jnp.full((LANES,), s, jnp.int32)
        dcol = jnp.full((LANES,), 8 + s, jnp.int32)
        mv = m1v[s, :]

        pltpu.sync_copy(z64.at[stripe], num_acc.at[stripe])
        for k in range(DEPTH):
            pltpu.sync_copy(z16.at[pl.ds(0, CH)], ee[k])
            issue(htab, k, k)
        plsc.subcore_barrier()

        @pl.loop(0, NCHUNK, step=DEPTH)
        def _slot(c0):
            for k in range(DEPTH):
                cc = c0 + k
                _drain(htab.at[pl.ds(0, CH)], hs[k], sem_g[k])
                _drain(a1.at[pl.ds(0, CH)], as_[k], sem_g[k])
                _drain(a1.at[pl.ds(0, CH)], ad[k], sem_g[k])

                @pl.loop(0, CH, step=LANES)
                def _group(g):
                    rows = iota + g
                    s0 = plsc.load_gather(as_[k], [rows, hcol])
                    d0 = plsc.load_gather(ad[k], [rows, dcol])
                    e0 = s0 + d0
                    e0 = jnp.maximum(e0, 0.2 * e0)
                    ee0 = jnp.exp(e0 - mv)
                    plsc.store_scatter(ee[k], [rows, hcol], ee0)

                    @pl.loop(0, LANES, step=4)
                    def _edges(t):
                        for u in range(4):
                            row = g + t + u
                            rv = jnp.full((LANES,), row, jnp.int32)
                            a0 = plsc.load_gather(ee[k], [rv, hcol])
                            for v in range(4):
                                sl = pl.ds(v * LANES, LANES)
                                hs[k][row, sl] = hs[k][row, sl] * a0

                pltpu.async_copy(hs[k], num_acc.at[idx_d.at[cc]], sem_s[k],
                                 add=True)
                pltpu.async_copy(ee[k], den_acc.at[idx_d.at[cc]], sem_s[k],
                                 add=True)
                b = (k + 3) % DEPTH
                tgt = cc + 3

                @pl.when(jnp.logical_and(tgt >= DEPTH, tgt < NCHUNK))
                def _():
                    _drain(htab.at[pl.ds(0, CH)], hs[b], sem_s[b])
                    _drain(a1.at[pl.ds(0, CH)], ee[b], sem_s[b])
                    issue(htab, tgt, b)

        for k in range(DEPTH):
            _drain(htab.at[pl.ds(0, CH)], hs[k], sem_s[k])
            _drain(a1.at[pl.ds(0, CH)], ee[k], sem_s[k])
        plsc.subcore_barrier()
        pltpu.sync_copy(num_acc.at[stripe],
                        ntabs[s].at[pl.ds(cid * NP + row0, STRIPE)])

    pltpu.sync_copy(den_acc.at[stripe],
                    denp.at[pl.ds(cid * NP + row0, STRIPE)])


def _sc1(hts, a1, m1t, src, dst, z64, z16):
    k = pl.kernel(
        _sc1_body,
        mesh=_MESH,
        out_type=[_f32(NSC * NP, 64)] * HEADS + [_f32(NSC * NP, 16)],
        scratch_types=[
            pltpu.VMEM((NCHUNK, CH), jnp.int32),
            pltpu.VMEM((NCHUNK, CH), jnp.int32),
        ]
        + [pltpu.VMEM((CH, 64), jnp.float32)] * DEPTH
        + [pltpu.VMEM((CH, 16), jnp.float32)] * DEPTH
        + [pltpu.VMEM((CH, 16), jnp.float32)] * DEPTH
        + [pltpu.VMEM((CH, 16), jnp.float32)] * DEPTH
        + [
            pltpu.VMEM((HEADS, 16), jnp.float32),
            pltpu.VMEM_SHARED((NP, 64), jnp.float32),
            pltpu.VMEM_SHARED((NP, 16), jnp.float32),
        ]
        + [pltpu.SemaphoreType.DMA] * (2 * DEPTH),
        compiler_params=_SC_PARAMS,
    )
    return k(*hts, a1, m1t, src, dst, z64, z16)


def _tc2_body(*refs):
    nrefs = refs[:HEADS]
    denp_ref, b1_ref, w2p_ref, r16_ref, h2p_ref, d2_ref, a2max_ref = \
        refs[HEADS:]
    i = pl.program_id(0)
    num = jnp.concatenate([nr[0] + nr[1] for nr in nrefs], axis=1)
    den = denp_ref[0] + denp_ref[1]
    dinv = jnp.where(den > 0.0, 1.0 / den, 0.0)
    scale = jnp.dot(dinv, r16_ref[...], preferred_element_type=jnp.float32)
    out1 = num * scale + b1_ref[...]
    act = jnp.where(out1 > 0.0, out1, jnp.exp(out1) - 1.0)
    h2p = jnp.dot(act, w2p_ref[...], preferred_element_type=jnp.float32)
    h2p_ref[...] = h2p
    d2_ref[...] = h2p[:, 32:48]
    cm = jnp.max(h2p, axis=0, keepdims=True)

    @pl.when(i == 0)
    def _():
        a2max_ref[...] = cm

    @pl.when(i > 0)
    def _():
        a2max_ref[...] = jnp.maximum(a2max_ref[...], cm)


def _tc2(ntabs, denp, b1r, W2p, R16):
    outs = [_f32(N, 48), _f32(N, 16), _f32(1, 48)]
    return pl.pallas_call(
        _tc2_body,
        grid=(GRID,),
        in_specs=[pl.BlockSpec((2, NB, 64), lambda i: (0, i, 0))] * HEADS
        + [
            pl.BlockSpec((2, NB, 16), lambda i: (0, i, 0)),
            pl.BlockSpec((1, 512), lambda i: (0, 0)),
            pl.BlockSpec((512, 48), lambda i: (0, 0)),
            pl.BlockSpec((16, 512), lambda i: (0, 0)),
        ],
        out_specs=[
            pl.BlockSpec((NB, 48), lambda i: (i, 0)),
            pl.BlockSpec((NB, 16), lambda i: (i, 0)),
            pl.BlockSpec((1, 48), lambda i: (0, 0)),
        ],
        out_shape=outs,
    )(*ntabs, denp, b1r, W2p, R16)


def _sc2_body(h2p, d2, m2t, src, dst, z48, z16,
              nump2, denp2,
              idx_s, idx_d, *rest):
    hs = rest[0:DEPTH]
    ad = rest[DEPTH:2 * DEPTH]
    ee = rest[2 * DEPTH:3 * DEPTH]
    m2v, num_acc, den_acc = rest[3 * DEPTH:3 * DEPTH + 3]
    sems = rest[3 * DEPTH + 3:]
    sem_g = sems[:DEPTH]
    sem_s = sems[DEPTH:]
    cid = lax.axis_index("c")
    sid = lax.axis_index("s")
    row0 = sid * STRIPE
    stripe = pl.ds(row0, STRIPE)

    pltpu.sync_copy(m2t, m2v)
    pltpu.sync_copy(src.at[cid].at[sid], idx_s)
    pltpu.sync_copy(dst.at[cid].at[sid], idx_d)
    pltpu.sync_copy(z48.at[stripe], num_acc.at[stripe])
    pltpu.sync_copy(z16.at[stripe], den_acc.at[stripe])

    def issue(cc, k):
        pltpu.async_copy(h2p.at[idx_s.at[cc]], hs[k], sem_g[k])
        pltpu.async_copy(d2.at[idx_d.at[cc]], ad[k], sem_g[k])

    for k in range(DEPTH):
        pltpu.sync_copy(z16.at[pl.ds(0, CH)], ee[k])
        issue(k, k)
    plsc.subcore_barrier()

    iota = lax.broadcasted_iota(jnp.int32, (LANES,), 0)
    col40 = jnp.full((LANES,), 40, jnp.int32)
    col9 = jnp.full((LANES,), 9, jnp.int32)
    col0 = jnp.full((LANES,), 0, jnp.int32)
    mv = m2v[0, :]

    @pl.loop(0, NCHUNK, step=DEPTH)
    def _slot(c0):
        for k in range(DEPTH):
            cc = c0 + k
            _drain(h2p.at[pl.ds(0, CH)], hs[k], sem_g[k])
            _drain(d2.at[pl.ds(0, CH)], ad[k], sem_g[k])

            @pl.loop(0, CH, step=LANES)
            def _group(g):
                rows = iota + g
                sv = plsc.load_gather(hs[k], [rows, col40])
                dv = plsc.load_gather(ad[k], [rows, col9])
                e = sv + dv
                e = jnp.maximum(e, 0.2 * e)
                eev = jnp.exp(e - mv)
                plsc.store_scatter(ee[k], [rows, col0], eev)

                @pl.loop(0, LANES, step=4)
                def _edges(t):
                    for u in range(4):
                        row = g + t + u
                        rv = jnp.full((LANES,), row, jnp.int32)
                        a0 = plsc.load_gather(ee[k], [rv, col0])
                        for v in range(3):
                            sl = pl.ds(v * LANES, LANES)
                            hs[k][row, sl] = hs[k][row, sl] * a0

            pltpu.async_copy(hs[k], num_acc.at[idx_d.at[cc]], sem_s[k],
                             add=True)
            pltpu.async_copy(ee[k], den_acc.at[idx_d.at[cc]], sem_s[k],
                             add=True)
            b = (k + 3) % DEPTH
            tgt = cc + 3

            @pl.when(jnp.logical_and(tgt >= DEPTH, tgt < NCHUNK))
            def _():
                _drain(h2p.at[pl.ds(0, CH)], hs[b], sem_s[b])
                _drain(d2.at[pl.ds(0, CH)], ee[b], sem_s[b])
                issue(tgt, b)

    for k in range(DEPTH):
        _drain(h2p.at[pl.ds(0, CH)], hs[k], sem_s[k])
        _drain(d2.at[pl.ds(0, CH)], ee[k], sem_s[k])
    plsc.subcore_barrier()
    pltpu.sync_copy(num_acc.at[stripe],
                    nump2.at[pl.ds(cid * NP + row0, STRIPE)])
    pltpu.sync_copy(den_acc.at[stripe],
                    denp2.at[pl.ds(cid * NP + row0, STRIPE)])


def _sc2(h2p, d2, m2t, src, dst, z48, z16):
    k = pl.kernel(
        _sc2_body,
        mesh=_MESH,
        out_type=[_f32(NSC * NP, 48), _f32(NSC * NP, 16)],
        scratch_types=[
            pltpu.VMEM((NCHUNK, CH), jnp.int32),
            pltpu.VMEM((NCHUNK, CH), jnp.int32),
        ]
        + [pltpu.VMEM((CH, 48), jnp.float32)] * DEPTH
        + [pltpu.VMEM((CH, 16), jnp.float32)] * DEPTH
        + [pltpu.VMEM((CH, 16), jnp.float32)] * DEPTH
        + [
            pltpu.VMEM((1, 16), jnp.float32),
            pltpu.VMEM_SHARED((NP, 48), jnp.float32),
            pltpu.VMEM_SHARED((NP, 16), jnp.float32),
        ]
        + [pltpu.SemaphoreType.DMA] * (2 * DEPTH),
        compiler_params=_SC_PARAMS,
    )
    return k(h2p, d2, m2t, src, dst, z48, z16)


def _tc3_body(nump2_ref, denp2_ref, b2_ref, out_ref):
    num = nump2_ref[0] + nump2_ref[1]
    den = denp2_ref[0] + denp2_ref[1]
    dinv = jnp.where(den > 0.0, 1.0 / den, 0.0)[:, 0:1]
    logits = num[:, 0:NCLS] * dinv + b2_ref[...]
    m = jnp.max(logits, axis=1, keepdims=True)
    z = jnp.exp(logits - m)
    lse = m + jnp.log(jnp.sum(z, axis=1, keepdims=True))
    out_ref[...] = logits - lse


def _tc3(nump2, denp2, b2r):
    return pl.pallas_call(
        _tc3_body,
        grid=(GRID,),
        in_specs=[
            pl.BlockSpec((2, NB, 48), lambda i: (0, i, 0)),
            pl.BlockSpec((2, NB, 16), lambda i: (0, i, 0)),
            pl.BlockSpec((1, NCLS), lambda i: (0, 0)),
        ],
        out_specs=pl.BlockSpec((NB, NCLS), lambda i: (i, 0)),
        out_shape=_f32(N, NCLS),
    )(nump2, denp2, b2r)


@jax.jit
def kernel(x, edge_index, W1, a_src1, a_dst1, b1, W2, a_src2, a_dst2, b2):
    src = edge_index[0].reshape(NSC, NTILE, NCHUNK, CH)
    dst = edge_index[1].reshape(NSC, NTILE, NCHUNK, CH)

    eye8 = jnp.eye(HEADS, dtype=jnp.float32)
    asrc_bd = (a_src1[:, :, None] * eye8[:, None, :]).reshape(HEADS * HID, HEADS)
    adst_bd = (a_dst1[:, :, None] * eye8[:, None, :]).reshape(HEADS * HID, HEADS)
    Abd1 = jnp.concatenate([asrc_bd, adst_bd], axis=1)
    W2p = jnp.concatenate(
        [W2, W2 @ a_src2.T, W2 @ a_dst2.T,
         jnp.zeros((HEADS * HID, 6), jnp.float32)], axis=1)
    R8 = jnp.repeat(eye8, HID, axis=1)
    R16 = jnp.concatenate([R8, jnp.zeros((8, 512), jnp.float32)], axis=0)
    b1r = b1.reshape(1, HEADS * HID)
    b2r = b2.reshape(1, NCLS)
    z64 = jnp.zeros((NP, 64), jnp.float32)
    z48 = jnp.zeros((NP, 48), jnp.float32)
    z16 = jnp.zeros((NP, 16), jnp.float32)

    tc1_out = _tc1(x, W1, Abd1)
    hts, a1, amax1 = tc1_out[:HEADS], tc1_out[HEADS], tc1_out[HEADS + 1]
    M1 = jnp.maximum(amax1[0, :8] + amax1[0, 8:], 0.0)
    m1t = jnp.broadcast_to(M1[:, None], (HEADS, 16))

    sc1_out = _sc1(hts, a1, m1t, src, dst, z64, z16)
    ntabs = [t.reshape(2, NP, 64) for t in sc1_out[:HEADS]]
    denp = sc1_out[HEADS].reshape(2, NP, 16)

    h2p, d2, a2max = _tc2(ntabs, denp, b1r, W2p, R16)
    M2 = jnp.maximum(a2max[0, 40] + a2max[0, 41], 0.0)
    m2t = jnp.full((1, 16), M2, jnp.float32)

    nump2, denp2 = _sc2(h2p, d2, m2t, src, dst, z48, z16)
    nump2 = nump2.reshape(2, NP, 48)
    denp2 = denp2.reshape(2, NP, 16)

    return _tc3(nump2, denp2, b2r)


# speedup vs baseline: 47.4539x; 1.0744x over previous
import dataclasses
import functools

import jax
import jax.numpy as jnp
from jax import lax
from jax.experimental import pallas as pl
from jax.experimental.pallas import tpu as pltpu
from jax.experimental.pallas import tpu_sc as plsc

N = 10000
E = 320000
F_IN = 128
HID = 64
HEADS = 8
NCLS = 40

NB = 1000
GRID = N // NB
NSC = 2
NTILE = 16
LANES = 16
EPC = E // NSC
EPT = EPC // NTILE
CH = 80
NP = 10240
STRIPE = NP // NTILE

_MESH = plsc.VectorSubcoreMesh(core_axis_name="c", subcore_axis_name="s")

_SC_PARAMS = pltpu.CompilerParams()
for _f, _v in (("needs_layout_passes", False), ("use_tc_tiling_on_sc", False)):
    if _f in pltpu.CompilerParams.__dataclass_fields__:
        _SC_PARAMS = dataclasses.replace(_SC_PARAMS, **{_f: _v})


def _f32(*shape):
    return jax.ShapeDtypeStruct(shape, jnp.float32)


def _tc1_body(x_ref, w1_ref, abd_ref, *outs):
    hrefs = outs[:HEADS]
    a1_ref, amax_ref = outs[HEADS:]
    i = pl.program_id(0)
    h = jnp.dot(x_ref[...], w1_ref[...], preferred_element_type=jnp.float32)
    for j in range(HEADS):
        hrefs[j][...] = h[:, 64 * j:64 * (j + 1)]
    a1 = jnp.dot(h, abd_ref[...], preferred_element_type=jnp.float32)
    a1_ref[...] = a1
    cm = jnp.max(a1, axis=0, keepdims=True)

    @pl.when(i == 0)
    def _():
        amax_ref[...] = cm

    @pl.when(i > 0)
    def _():
        amax_ref[...] = jnp.maximum(amax_ref[...], cm)


def _tc1(x, W1, Abd1):
    outs = [_f32(N, 64)] * HEADS + [_f32(N, 16), _f32(1, 16)]
    return pl.pallas_call(
        _tc1_body,
        grid=(GRID,),
        in_specs=[
            pl.BlockSpec((NB, F_IN), lambda i: (i, 0)),
            pl.BlockSpec((F_IN, 512), lambda i: (0, 0)),
            pl.BlockSpec((512, 16), lambda i: (0, 0)),
        ],
        out_specs=[pl.BlockSpec((NB, 64), lambda i: (i, 0))] * HEADS
        + [
            pl.BlockSpec((NB, 16), lambda i: (i, 0)),
            pl.BlockSpec((1, 16), lambda i: (0, 0)),
        ],
        out_shape=outs,
    )(x, W1, Abd1)


NCHUNK = EPT // CH
DEPTH = 5


def _drain(dummy_src, dst, sem):
    pltpu.make_async_copy(dummy_src, dst, sem).wait()


def _sc0_body(a1, m1t, src, dst, z16, eeh, denp, idx_s, idx_d, *rest):
    as_ = rest[0:DEPTH]
    ad = rest[DEPTH:2 * DEPTH]
    ee = rest[2 * DEPTH:3 * DEPTH]
    m1v, eebig, den_acc = rest[3 * DEPTH:3 * DEPTH + 3]
    sems = rest[3 * DEPTH + 3:]
    sem_g = sems[:DEPTH]
    sem_s = sems[DEPTH:]
    cid = lax.axis_index("c")
    sid = lax.axis_index("s")
    row0 = sid * STRIPE
    stripe = pl.ds(row0, STRIPE)
    ebase = (cid * NTILE + sid) * EPT

    pltpu.sync_copy(m1t, m1v)
    pltpu.sync_copy(z16.at[stripe], den_acc.at[stripe])
    pltpu.sync_copy(src.at[cid].at[sid], idx_s)
    pltpu.sync_copy(dst.at[cid].at[sid], idx_d)

    iota = lax.broadcasted_iota(jnp.int32, (LANES,), 0)
    mv = [m1v[j, :] for j in range(HEADS)]

    def issue(cc, k):
        pltpu.async_copy(a1.at[idx_s.at[cc]], as_[k], sem_g[k])
        pltpu.async_copy(a1.at[idx_d.at[cc]], ad[k], sem_g[k])

    for k in range(DEPTH):
        pltpu.sync_copy(z16.at[pl.ds(0, CH)], ee[k])
        issue(k, k)
    plsc.subcore_barrier()

    @pl.loop(0, NCHUNK, step=DEPTH)
    def _slot(c0):
        for k in range(DEPTH):
            cc = c0 + k
            _drain(a1.at[pl.ds(0, CH)], as_[k], sem_g[k])
            _drain(a1.at[pl.ds(0, CH)], ad[k], sem_g[k])

            @pl.loop(0, CH, step=LANES)
            def _group(g):
                rows = iota + g
                off = cc * CH + g
                for j in range(HEADS):
                    jc = jnp.full((LANES,), j, jnp.int32)
                    jd = jnp.full((LANES,), 8 + j, jnp.int32)
                    s0 = plsc.load_gather(as_[k], [rows, jc])
                    d0 = plsc.load_gather(ad[k], [rows, jd])
                    e0 = s0 + d0
                    e0 = jnp.maximum(e0, 0.2 * e0)
                    ee0 = jnp.exp(e0 - mv[j])
                    plsc.store_scatter(ee[k], [rows, jc], ee0)
                    eebig[j, pl.ds(off, LANES)] = ee0

            pltpu.async_copy(ee[k], den_acc.at[idx_d.at[cc]], sem_s[k],
                             add=True)
            b = (k + 3) % DEPTH
            tgt = cc + 3

            @pl.when(jnp.logical_and(tgt >= DEPTH, tgt < NCHUNK))
            def _():
                _drain(a1.at[pl.ds(0, CH)], ee[b], sem_s[b])
                issue(tgt, b)

    for k in range(DEPTH):
        _drain(a1.at[pl.ds(0, CH)], ee[k], sem_s[k])
    for j in range(HEADS):
        pltpu.sync_copy(eebig.at[j], eeh.at[j].at[pl.ds(ebase, EPT)])
    plsc.subcore_barrier()
    pltpu.sync_copy(den_acc.at[stripe],
                    denp.at[pl.ds(cid * NP + row0, STRIPE)])


def _sc0(a1, m1t, src, dst, z16):
    k = pl.kernel(
        _sc0_body,
        mesh=_MESH,
        out_type=[_f32(HEADS, E), _f32(NSC * NP, 16)],
        scratch_types=[
            pltpu.VMEM((NCHUNK, CH), jnp.int32),
            pltpu.VMEM((NCHUNK, CH), jnp.int32),
        ]
        + [pltpu.VMEM((CH, 16), jnp.float32)] * (3 * DEPTH)
        + [
            pltpu.VMEM((HEADS, 16), jnp.float32),
            pltpu.VMEM((HEADS, EPT), jnp.float32),
            pltpu.VMEM_SHARED((NP, 16), jnp.float32),
        ]
        + [pltpu.SemaphoreType.DMA] * (2 * DEPTH),
        compiler_params=_SC_PARAMS,
    )
    return k(a1, m1t, src, dst, z16)


def _sc1_body(*args):
    htabs = args[0:HEADS]
    eeh, src, dst, z64 = args[HEADS:HEADS + 4]
    ntabs = args[HEADS + 4:2 * HEADS + 4]
    idx_s, idx_d = args[2 * HEADS + 4:2 * HEADS + 6]
    rest = args[2 * HEADS + 6:]
    hs = rest[0:DEPTH]
    eet = rest[DEPTH]
    num_acc = rest[DEPTH + 1]
    sems = rest[DEPTH + 2:]
    sem_g = sems[:DEPTH]
    sem_s = sems[DEPTH:]
    cid = lax.axis_index("c")
    sid = lax.axis_index("s")
    row0 = sid * STRIPE
    stripe = pl.ds(row0, STRIPE)
    ebase = (cid * NTILE + sid) * EPT

    pltpu.sync_copy(src.at[cid].at[sid], idx_s)
    pltpu.sync_copy(dst.at[cid].at[sid], idx_d)

    def issue(htab, cc, k):
        pltpu.async_copy(htab.at[idx_s.at[cc]], hs[k], sem_g[k])

    for s in range(HEADS):
        htab = htabs[s]

        pltpu.sync_copy(z64.at[stripe], num_acc.at[stripe])
        pltpu.sync_copy(eeh.at[s].at[pl.ds(ebase, EPT)], eet)
        for k in range(DEPTH):
            issue(htab, k, k)
        plsc.subcore_barrier()

        @pl.loop(0, NCHUNK, step=DEPTH)
        def _slot(c0):
            for k in range(DEPTH):
                cc = c0 + k
                _drain(htab.at[pl.ds(0, CH)], hs[k], sem_g[k])

                @pl.loop(0, CH, step=4)
                def _edges(t):
                    for u in range(4):
                        row = t + u
                        rv = jnp.full((LANES,), cc * CH + row, jnp.int32)
                        a0 = plsc.load_gather(eet, [rv])
                        for v in range(4):
                            sl = pl.ds(v * LANES, LANES)
                            hs[k][row, sl] = hs[k][row, sl] * a0

                pltpu.async_copy(hs[k], num_acc.at[idx_d.at[cc]], sem_s[k],
                                 add=True)
                b = (k + 3) % DEPTH
                tgt = cc + 3

                @pl.when(jnp.logical_and(tgt >= DEPTH, tgt < NCHUNK))
                def _():
                    _drain(htab.at[pl.ds(0, CH)], hs[b], sem_s[b])
                    issue(htab, tgt, b)

        for k in range(DEPTH):
            _drain(htab.at[pl.ds(0, CH)], hs[k], sem_s[k])
        plsc.subcore_barrier()
        pltpu.sync_copy(num_acc.at[stripe],
                        ntabs[s].at[pl.ds(cid * NP + row0, STRIPE)])


def _sc1(hts, eeh, src, dst, z64):
    k = pl.kernel(
        _sc1_body,
        mesh=_MESH,
        out_type=[_f32(NSC * NP, 64)] * HEADS,
        scratch_types=[
            pltpu.VMEM((NCHUNK, CH), jnp.int32),
            pltpu.VMEM((NCHUNK, CH), jnp.int32),
        ]
        + [pltpu.VMEM((CH, 64), jnp.float32)] * DEPTH
        + [
            pltpu.VMEM((EPT,), jnp.float32),
            pltpu.VMEM_SHARED((NP, 64), jnp.float32),
        ]
        + [pltpu.SemaphoreType.DMA] * (2 * DEPTH),
        compiler_params=_SC_PARAMS,
    )
    return k(*hts, eeh, src, dst, z64)


def _tc2_body(*refs):
    nrefs = refs[:HEADS]
    denp_ref, b1_ref, w2p_ref, r16_ref, h2p_ref, d2_ref, a2max_ref = \
        refs[HEADS:]
    i = pl.program_id(0)
    num = jnp.concatenate([nr[0] + nr[1] for nr in nrefs], axis=1)
    den = denp_ref[0] + denp_ref[1]
    dinv = jnp.where(den > 0.0, 1.0 / den, 0.0)
    scale = jnp.dot(dinv, r16_ref[...], preferred_element_type=jnp.float32)
    out1 = num * scale + b1_ref[...]
    act = jnp.where(out1 > 0.0, out1, jnp.exp(out1) - 1.0)
    h2p = jnp.dot(act, w2p_ref[...], preferred_element_type=jnp.float32)
    h2p_ref[...] = h2p
    d2_ref[...] = h2p[:, 32:48]
    cm = jnp.max(h2p, axis=0, keepdims=True)

    @pl.when(i == 0)
    def _():
        a2max_ref[...] = cm

    @pl.when(i > 0)
    def _():
        a2max_ref[...] = jnp.maximum(a2max_ref[...], cm)


def _tc2(ntabs, denp, b1r, W2p, R16):
    outs = [_f32(N, 48), _f32(N, 16), _f32(1, 48)]
    return pl.pallas_call(
        _tc2_body,
        grid=(GRID,),
        in_specs=[pl.BlockSpec((2, NB, 64), lambda i: (0, i, 0))] * HEADS
        + [
            pl.BlockSpec((2, NB, 16), lambda i: (0, i, 0)),
            pl.BlockSpec((1, 512), lambda i: (0, 0)),
            pl.BlockSpec((512, 48), lambda i: (0, 0)),
            pl.BlockSpec((16, 512), lambda i: (0, 0)),
        ],
        out_specs=[
            pl.BlockSpec((NB, 48), lambda i: (i, 0)),
            pl.BlockSpec((NB, 16), lambda i: (i, 0)),
            pl.BlockSpec((1, 48), lambda i: (0, 0)),
        ],
        out_shape=outs,
    )(*ntabs, denp, b1r, W2p, R16)


def _sc2_body(h2p, d2, m2t, src, dst, z48, z16,
              nump2, denp2,
              idx_s, idx_d, *rest):
    hs = rest[0:DEPTH]
    ad = rest[DEPTH:2 * DEPTH]
    ee = rest[2 * DEPTH:3 * DEPTH]
    m2v, num_acc, den_acc = rest[3 * DEPTH:3 * DEPTH + 3]
    sems = rest[3 * DEPTH + 3:]
    sem_g = sems[:DEPTH]
    sem_s = sems[DEPTH:]
    cid = lax.axis_index("c")
    sid = lax.axis_index("s")
    row0 = sid * STRIPE
    stripe = pl.ds(row0, STRIPE)

    pltpu.sync_copy(m2t, m2v)
    pltpu.sync_copy(src.at[cid].at[sid], idx_s)
    pltpu.sync_copy(dst.at[cid].at[sid], idx_d)
    pltpu.sync_copy(z48.at[stripe], num_acc.at[stripe])
    pltpu.sync_copy(z16.at[stripe], den_acc.at[stripe])

    def issue(cc, k):
        pltpu.async_copy(h2p.at[idx_s.at[cc]], hs[k], sem_g[k])
        pltpu.async_copy(d2.at[idx_d.at[cc]], ad[k], sem_g[k])

    for k in range(DEPTH):
        pltpu.sync_copy(z16.at[pl.ds(0, CH)], ee[k])
        issue(k, k)
    plsc.subcore_barrier()

    iota = lax.broadcasted_iota(jnp.int32, (LANES,), 0)
    col40 = jnp.full((LANES,), 40, jnp.int32)
    col9 = jnp.full((LANES,), 9, jnp.int32)
    col0 = jnp.full((LANES,), 0, jnp.int32)
    mv = m2v[0, :]

    @pl.loop(0, NCHUNK, step=DEPTH)
    def _slot(c0):
        for k in range(DEPTH):
            cc = c0 + k
            _drain(h2p.at[pl.ds(0, CH)], hs[k], sem_g[k])
            _drain(d2.at[pl.ds(0, CH)], ad[k], sem_g[k])

            @pl.loop(0, CH, step=LANES)
            def _group(g):
                rows = iota + g
                sv = plsc.load_gather(hs[k], [rows, col40])
                dv = plsc.load_gather(ad[k], [rows, col9])
                e = sv + dv
                e = jnp.maximum(e, 0.2 * e)
                eev = jnp.exp(e - mv)
                plsc.store_scatter(ee[k], [rows, col0], eev)

                @pl.loop(0, LANES, step=4)
                def _edges(t):
                    for u in range(4):
                        row = g + t + u
                        rv = jnp.full((LANES,), row, jnp.int32)
                        a0 = plsc.load_gather(ee[k], [rv, col0])
                        for v in range(3):
                            sl = pl.ds(v * LANES, LANES)
                            hs[k][row, sl] = hs[k][row, sl] * a0

            pltpu.async_copy(hs[k], num_acc.at[idx_d.at[cc]], sem_s[k],
                             add=True)
            pltpu.async_copy(ee[k], den_acc.at[idx_d.at[cc]], sem_s[k],
                             add=True)
            b = (k + 3) % DEPTH
            tgt = cc + 3

            @pl.when(jnp.logical_and(tgt >= DEPTH, tgt < NCHUNK))
            def _():
                _drain(h2p.at[pl.ds(0, CH)], hs[b], sem_s[b])
                _drain(d2.at[pl.ds(0, CH)], ee[b], sem_s[b])
                issue(tgt, b)

    for k in range(DEPTH):
        _drain(h2p.at[pl.ds(0, CH)], hs[k], sem_s[k])
        _drain(d2.at[pl.ds(0, CH)], ee[k], sem_s[k])
    plsc.subcore_barrier()
    pltpu.sync_copy(num_acc.at[stripe],
                    nump2.at[pl.ds(cid * NP + row0, STRIPE)])
    pltpu.sync_copy(den_acc.at[stripe],
                    denp2.at[pl.ds(cid * NP + row0, STRIPE)])


def _sc2(h2p, d2, m2t, src, dst, z48, z16):
    k = pl.kernel(
        _sc2_body,
        mesh=_MESH,
        out_type=[_f32(NSC * NP, 48), _f32(NSC * NP, 16)],
        scratch_types=[
            pltpu.VMEM((NCHUNK, CH), jnp.int32),
            pltpu.VMEM((NCHUNK, CH), jnp.int32),
        ]
        + [pltpu.VMEM((CH, 48), jnp.float32)] * DEPTH
        + [pltpu.VMEM((CH, 16), jnp.float32)] * DEPTH
        + [pltpu.VMEM((CH, 16), jnp.float32)] * DEPTH
        + [
            pltpu.VMEM((1, 16), jnp.float32),
            pltpu.VMEM_SHARED((NP, 48), jnp.float32),
            pltpu.VMEM_SHARED((NP, 16), jnp.float32),
        ]
        + [pltpu.SemaphoreType.DMA] * (2 * DEPTH),
        compiler_params=_SC_PARAMS,
    )
    return k(h2p, d2, m2t, src, dst, z48, z16)


def _tc3_body(nump2_ref, denp2_ref, b2_ref, out_ref):
    num = nump2_ref[0] + nump2_ref[1]
    den = denp2_ref[0] + denp2_ref[1]
    dinv = jnp.where(den > 0.0, 1.0 / den, 0.0)[:, 0:1]
    logits = num[:, 0:NCLS] * dinv + b2_ref[...]
    m = jnp.max(logits, axis=1, keepdims=True)
    z = jnp.exp(logits - m)
    lse = m + jnp.log(jnp.sum(z, axis=1, keepdims=True))
    out_ref[...] = logits - lse


def _tc3(nump2, denp2, b2r):
    return pl.pallas_call(
        _tc3_body,
        grid=(GRID,),
        in_specs=[
            pl.BlockSpec((2, NB, 48), lambda i: (0, i, 0)),
            pl.BlockSpec((2, NB, 16), lambda i: (0, i, 0)),
            pl.BlockSpec((1, NCLS), lambda i: (0, 0)),
        ],
        out_specs=pl.BlockSpec((NB, NCLS), lambda i: (i, 0)),
        out_shape=_f32(N, NCLS),
    )(nump2, denp2, b2r)


@jax.jit
def kernel(x, edge_index, W1, a_src1, a_dst1, b1, W2, a_src2, a_dst2, b2):
    src = edge_index[0].reshape(NSC, NTILE, NCHUNK, CH)
    dst = edge_index[1].reshape(NSC, NTILE, NCHUNK, CH)

    eye8 = jnp.eye(HEADS, dtype=jnp.float32)
    asrc_bd = (a_src1[:, :, None] * eye8[:, None, :]).reshape(HEADS * HID, HEADS)
    adst_bd = (a_dst1[:, :, None] * eye8[:, None, :]).reshape(HEADS * HID, HEADS)
    Abd1 = jnp.concatenate([asrc_bd, adst_bd], axis=1)
    W2p = jnp.concatenate(
        [W2, W2 @ a_src2.T, W2 @ a_dst2.T,
         jnp.zeros((HEADS * HID, 6), jnp.float32)], axis=1)
    R8 = jnp.repeat(eye8, HID, axis=1)
    R16 = jnp.concatenate([R8, jnp.zeros((8, 512), jnp.float32)], axis=0)
    b1r = b1.reshape(1, HEADS * HID)
    b2r = b2.reshape(1, NCLS)
    z64 = jnp.zeros((NP, 64), jnp.float32)
    z48 = jnp.zeros((NP, 48), jnp.float32)
    z16 = jnp.zeros((NP, 16), jnp.float32)

    tc1_out = _tc1(x, W1, Abd1)
    hts, a1, amax1 = tc1_out[:HEADS], tc1_out[HEADS], tc1_out[HEADS + 1]
    M1 = jnp.maximum(amax1[0, :8] + amax1[0, 8:], 0.0)
    m1t = jnp.broadcast_to(M1[:, None], (HEADS, 16))

    eeh, denp = _sc0(a1, m1t, src, dst, z16)
    denp = denp.reshape(2, NP, 16)
    ntabs = [t.reshape(2, NP, 64) for t in _sc1(hts, eeh, src, dst, z64)]

    h2p, d2, a2max = _tc2(ntabs, denp, b1r, W2p, R16)
    M2 = jnp.maximum(a2max[0, 40] + a2max[0, 41], 0.0)
    m2t = jnp.full((1, 16), M2, jnp.float32)

    nump2, denp2 = _sc2(h2p, d2, m2t, src, dst, z48, z16)
    nump2 = nump2.reshape(2, NP, 48)
    denp2 = denp2.reshape(2, NP, 16)

    return _tc3(nump2, denp2, b2r)
